# Initial kernel scaffold; baseline (speedup 1.0000x reference)
import jax
import jax.numpy as jnp
from jax.experimental import pallas as pl


def kernel(x, indices, W, b):
    raise NotImplementedError("write your pallas kernel here")


# speedup vs baseline: 1.7432x; 1.7432x over previous
import functools

import jax
import jax.numpy as jnp
from jax import lax
from jax.experimental import pallas as pl
from jax.experimental.pallas import tpu as pltpu
from jax.experimental.pallas import tpu_sc as plsc

N_NODES = 10000
SEQ = 32
CH = 128
M_PAD = 10240

TC_BM = 256
TC_GRID = M_PAD // TC_BM

NW = 32
NODES_PER_W = M_PAD // NW
NODES_PER_CHUNK = 4
CHUNKS = NODES_PER_W // NODES_PER_CHUNK
IDX_PER_CHUNK = NODES_PER_CHUNK * SEQ


def _zk_body(x_ref, w_ref, o_ref):
    acc = lax.dot_general(
        x_ref[...], w_ref[...], (((1,), (0,)), ((), ())),
        preferred_element_type=jnp.float32)
    for j in range(SEQ):
        o_ref[:, j, :] = acc[:, CH * j:CH * (j + 1)]


def _z_transform(x_pad, w4):
    return pl.pallas_call(
        _zk_body,
        grid=(TC_GRID,),
        in_specs=[
            pl.BlockSpec((TC_BM, CH), lambda i: (i, 0)),
            pl.BlockSpec((CH, SEQ * CH), lambda i: (0, 0)),
        ],
        out_specs=pl.BlockSpec((TC_BM, SEQ, CH), lambda i: (i, 0, 0)),
        out_shape=jax.ShapeDtypeStruct((M_PAD, SEQ, CH), jnp.float32),
    )(x_pad, w4)


def _bag_body(z_ref, idx_ref, b_ref, o_ref, idxv, bv, gbuf, obuf, semg, semo):
    wid = lax.axis_index("s") * 2 + lax.axis_index("c")

    pltpu.sync_copy(idx_ref.at[wid], idxv)
    pltpu.sync_copy(b_ref, bv)

    def start_gather(c, buf):
        pltpu.async_copy(z_ref.at[idxv.at[c]], gbuf.at[buf], semg.at[buf])

    def wait_gather(c, buf):
        pltpu.make_async_copy(z_ref.at[idxv.at[c]], gbuf.at[buf],
                              semg.at[buf]).wait()

    def out_rows(c):
        return o_ref.at[pl.ds(wid * NODES_PER_W + c * NODES_PER_CHUNK,
                              NODES_PER_CHUNK)]

    start_gather(0, 0)

    def step(i, _):
        for buf in range(2):
            c = i * 2 + buf

            @pl.when(c + 1 < CHUNKS)
            def _():
                start_gather(c + 1, 1 - buf)

            wait_gather(c, buf)

            @pl.when(c >= 2)
            def _():
                pltpu.make_async_copy(obuf.at[buf], out_rows(c),
                                      semo.at[buf]).wait()

            for q in range(NODES_PER_CHUNK):
                def rbody(r, acc, _q=q):
                    return tuple(
                        a + gbuf[buf, _q * SEQ + r, pl.ds(16 * v, 16)]
                        for v, a in enumerate(acc))
                acc0 = tuple(bv[pl.ds(16 * v, 16)] for v in range(8))
                acc = lax.fori_loop(0, SEQ, rbody, acc0)
                for v in range(8):
                    obuf[buf, q, pl.ds(16 * v, 16)] = acc[v]

            pltpu.async_copy(obuf.at[buf], out_rows(c), semo.at[buf])
        return _

    lax.fori_loop(0, CHUNKS // 2, step, None)

    for buf in range(2):
        c = CHUNKS - 2 + buf
        pltpu.make_async_copy(obuf.at[buf], out_rows(c), semo.at[buf]).wait()


_bag = pl.kernel(
    _bag_body,
    out_type=jax.ShapeDtypeStruct((M_PAD, CH), jnp.float32),
    mesh=plsc.VectorSubcoreMesh(core_axis_name="c", subcore_axis_name="s"),
    scratch_types=[
        pltpu.VMEM((CHUNKS, IDX_PER_CHUNK), jnp.int32),
        pltpu.VMEM((CH,), jnp.float32),
        pltpu.VMEM((2, IDX_PER_CHUNK, CH), jnp.float32),
        pltpu.VMEM((2, NODES_PER_CHUNK, CH), jnp.float32),
        pltpu.SemaphoreType.DMA((2,)),
        pltpu.SemaphoreType.DMA((2,)),
    ],
)


def kernel(x, indices, W, b):
    idx32 = indices.astype(jnp.int32)
    jj = jnp.arange(SEQ, dtype=jnp.int32)[None, :]
    flat = idx32 * SEQ + jj
    flat = jnp.pad(flat, ((0, M_PAD - N_NODES), (0, 0)))
    flat = flat.reshape(NW, CHUNKS, IDX_PER_CHUNK)

    x_pad = jnp.pad(x, ((0, M_PAD - N_NODES), (0, 0)))
    w4 = W.reshape(CH, SEQ, CH).transpose(2, 1, 0).reshape(CH, SEQ * CH)

    z3 = _z_transform(x_pad, w4)
    zf = z3.reshape(M_PAD * SEQ, CH)

    out = _bag(zf, flat, b)
    return out[:N_NODES]
